# Initial kernel scaffold; baseline (speedup 1.0000x reference)
import jax
import jax.numpy as jnp
from jax.experimental import pallas as pl


def kernel(x, edge_connected_to, edge_ordered_next, edge_represents, edge_represented_by, edge_neighboring_vertical, edge_neighboring_horizontal, edge_contains, edge_order, edge_perpendicular, W0_connected_to, W0_ordered_next, W0_root, W1_connected_to, W1_ordered_next, W1_root, W1_skip, W2_represents, W2_represented_by, W2_neighboring_vertical, W2_neighboring_horizontal, W2_contains, W2_order, W2_perpendicular, W2_root, W2_skip):
    raise NotImplementedError("write your pallas kernel here")


# speedup vs baseline: 2.5075x; 2.5075x over previous
import functools

import jax
import jax.numpy as jnp
from jax import lax
from jax.experimental import pallas as pl
from jax.experimental.pallas import tpu as pltpu
from jax.experimental.pallas import tpu_sc as plsc

N = 50000
E = 100000

NC = 2
NS = 16
L = 16

TPT = 3136
NPAD = NS * TPT
ZR = 196
GH = 28
G = 128
GPT = 56
NG = NS * GPT
EPAD = NG * G


def _sc_scatter(feat, table, jobs):
    njobs = len(jobs)
    rounds = []
    for r in range((njobs + 1) // 2):
        c0 = (2 * r, jobs[2 * r])
        c1 = (2 * r + 1, jobs[2 * r + 1]) if 2 * r + 1 < njobs else None
        rounds.append((c0, c1))

    idx_inputs = []
    job_idx_pos = []
    for src, dst in jobs:
        if src is None:
            job_idx_pos.append((None, len(idx_inputs)))
            idx_inputs.append(dst)
        else:
            job_idx_pos.append((len(idx_inputs), len(idx_inputs) + 1))
            idx_inputs.append(src)
            idx_inputs.append(dst)

    n_idx = len(idx_inputs)
    mesh = plsc.VectorSubcoreMesh(core_axis_name="c", subcore_axis_name="s")

    @functools.partial(
        pl.kernel,
        out_type=[jax.ShapeDtypeStruct((NPAD, feat), jnp.float32)] * njobs,
        mesh=mesh,
        scratch_types=[
            pltpu.MemorySpace.VMEM_SHARED((NPAD, feat), jnp.float32),
            pltpu.MemorySpace.VMEM((GH, G), jnp.int32),
            pltpu.MemorySpace.VMEM((GH, G), jnp.int32),
            pltpu.MemorySpace.VMEM((G, feat), jnp.float32),
            pltpu.MemorySpace.VMEM((ZR, feat), jnp.float32),
            pltpu.MemorySpace.VMEM((ZR, feat), jnp.float32),
            pltpu.SemaphoreType.DMA,
        ],
        compiler_params=pltpu.CompilerParams(use_tc_tiling_on_sc=False),
    )
    def run(*refs):
        table_ref = refs[0]
        idx_refs = refs[1:1 + n_idx]
        out_refs = refs[1 + n_idx:1 + n_idx + njobs]
        acc, sidx, didx, rows, zbuf, fbuf, sem = refs[1 + n_idx + njobs:]

        c = lax.axis_index("c")
        s = lax.axis_index("s")
        base = s * TPT
        gbase = s * GPT

        zero16 = jnp.zeros((L,), jnp.float32)
        one16 = jnp.ones((L,), jnp.float32)

        def fill_zbuf(i, carry):
            for k in range(feat // L):
                zbuf[i, pl.ds(k * L, L)] = zero16
            return carry

        lax.fori_loop(0, ZR, fill_zbuf, 0)

        def emit_job(job_id):
            src_pos, dst_pos = job_idx_pos[job_id]
            out_ref = out_refs[job_id]

            def zero_acc():
                for k in range(TPT // ZR):
                    pltpu.sync_copy(zbuf, acc.at[pl.ds(base + k * ZR, ZR)])

            def accumulate():
                if src_pos is None:
                    def fill_ones(i, carry):
                        for k in range(feat // L):
                            rows[i, pl.ds(k * L, L)] = one16
                        return carry

                    lax.fori_loop(0, G, fill_ones, 0)

                for half in range(GPT // GH):
                    hb = gbase + half * GH
                    pltpu.sync_copy(idx_refs[dst_pos].at[pl.ds(hb, GH)],
                                    didx)
                    if src_pos is not None:
                        pltpu.sync_copy(idx_refs[src_pos].at[pl.ds(hb, GH)],
                                        sidx)

                        def grp(g, carry):
                            pltpu.async_copy(table_ref.at[sidx.at[g]], rows,
                                             sem).wait()
                            pltpu.sync_copy(rows, acc.at[didx.at[g]],
                                            add=True)
                            return carry

                        lax.fori_loop(0, GH, grp, 0)
                    else:
                        def grp(g, carry):
                            pltpu.sync_copy(rows, acc.at[didx.at[g]],
                                            add=True)
                            return carry

                        lax.fori_loop(0, GH, grp, 0)

            def flush():
                for k in range(TPT // ZR):
                    sl = pl.ds(base + k * ZR, ZR)
                    pltpu.sync_copy(acc.at[sl], fbuf)
                    pltpu.sync_copy(fbuf, out_ref.at[sl])

            return zero_acc, accumulate, flush

        for c0_job, c1_job in rounds:
            stages0 = emit_job(c0_job[0])
            stages1 = emit_job(c1_job[0]) if c1_job is not None else None
            pl.when(c == 0)(stages0[0])
            if stages1 is not None:
                pl.when(c == 1)(stages1[0])
            plsc.subcore_barrier()
            pl.when(c == 0)(stages0[1])
            if stages1 is not None:
                pl.when(c == 1)(stages1[1])
            plsc.subcore_barrier()
            pl.when(c == 0)(stages0[2])
            if stages1 is not None:
                pl.when(c == 1)(stages1[2])
            plsc.subcore_barrier()

    return run(table, *idx_inputs)


BLK = 1024
GRID = NPAD // BLK


def _row_spec(f):
    return pl.BlockSpec((BLK, f), lambda i: (i, 0))


def _full_spec(fi, fo):
    return pl.BlockSpec((fi, fo), lambda i: (0, 0))


def _tc_layer0(x, s_ct, s_on, w_r, w_ct, w_on):
    def body(x_ref, a_ref, b_ref, wr_ref, wa_ref, wb_ref, o_ref):
        acc = jnp.dot(x_ref[...], wr_ref[...], preferred_element_type=jnp.float32)
        acc += jnp.dot(a_ref[...], wa_ref[...], preferred_element_type=jnp.float32)
        acc += jnp.dot(b_ref[...], wb_ref[...], preferred_element_type=jnp.float32)
        o_ref[...] = acc

    return pl.pallas_call(
        body,
        grid=(GRID,),
        in_specs=[_row_spec(16), _row_spec(16), _row_spec(16),
                  _full_spec(16, 16), _full_spec(16, 16), _full_spec(16, 16)],
        out_specs=_row_spec(16),
        out_shape=jax.ShapeDtypeStruct((NPAD, 16), jnp.float32),
    )(x, s_ct, s_on, w_r, w_ct, w_on)


def _tc_layer1(h, s_ct, s_on, w_r, w_skip, w_ct, w_on):
    def body(h_ref, a_ref, b_ref, wr_ref, ws_ref, wa_ref, wb_ref, o_ref):
        acc = jnp.dot(h_ref[...], wr_ref[...] + ws_ref[...],
                      preferred_element_type=jnp.float32)
        acc += jnp.dot(a_ref[...], wa_ref[...], preferred_element_type=jnp.float32)
        acc += jnp.dot(b_ref[...], wb_ref[...], preferred_element_type=jnp.float32)
        o_ref[...] = acc

    return pl.pallas_call(
        body,
        grid=(GRID,),
        in_specs=[_row_spec(16), _row_spec(16), _row_spec(16),
                  _full_spec(16, 32), _full_spec(16, 32),
                  _full_spec(16, 32), _full_spec(16, 32)],
        out_specs=_row_spec(32),
        out_shape=jax.ShapeDtypeStruct((NPAD, 32), jnp.float32),
    )(h, s_ct, s_on, w_r, w_skip, w_ct, w_on)


MEAN_IDX = (2, 3, 6)


def _tc_layer2(h1, s_list, cnt_list, w_r, w_skip, w_list):
    def body(*refs):
        h_ref = refs[0]
        s_refs = refs[1:8]
        c_refs = refs[8:11]
        wr_ref, ws_ref = refs[11], refs[12]
        w_refs = refs[13:20]
        o_ref = refs[20]
        acc = jnp.dot(h_ref[...], wr_ref[...] + ws_ref[...],
                      preferred_element_type=jnp.float32)
        ci = 0
        for i in range(7):
            m = s_refs[i][...]
            if i in MEAN_IDX:
                cnt = c_refs[ci][...][:, 0:1]
                m = m / jnp.maximum(cnt, 1.0)
                ci += 1
            acc += jnp.dot(m, w_refs[i][...],
                           preferred_element_type=jnp.float32)
        o_ref[...] = jnp.maximum(acc, 0.0)

    return pl.pallas_call(
        body,
        grid=(GRID,),
        in_specs=([_row_spec(32)] + [_row_spec(32)] * 7 + [_row_spec(16)] * 3
                  + [_full_spec(32, 64)] * 9),
        out_specs=_row_spec(64),
        out_shape=jax.ShapeDtypeStruct((NPAD, 64), jnp.float32),
    )(h1, *s_list, *cnt_list, w_r, w_skip, *w_list)


def _prep_edges(ei):
    pad = jnp.full((EPAD - E,), N, jnp.int32)
    src = jnp.concatenate([ei[0], pad]).reshape(NG, G)
    dst = jnp.concatenate([ei[1], pad]).reshape(NG, G)
    return src, dst


def _pad_w(w):
    return jnp.zeros((16, w.shape[1]), jnp.float32).at[:w.shape[0]].set(w)


def kernel(x, edge_connected_to, edge_ordered_next, edge_represents,
           edge_represented_by, edge_neighboring_vertical,
           edge_neighboring_horizontal, edge_contains, edge_order,
           edge_perpendicular, W0_connected_to, W0_ordered_next, W0_root,
           W1_connected_to, W1_ordered_next, W1_root, W1_skip,
           W2_represents, W2_represented_by, W2_neighboring_vertical,
           W2_neighboring_horizontal, W2_contains, W2_order,
           W2_perpendicular, W2_root, W2_skip):
    x_pad = jnp.zeros((NPAD, 16), jnp.float32).at[:N, :6].set(x)

    ct = _prep_edges(edge_connected_to)
    on = _prep_edges(edge_ordered_next)
    l2_edges = [_prep_edges(e) for e in (
        edge_represents, edge_represented_by, edge_neighboring_vertical,
        edge_neighboring_horizontal, edge_contains, edge_order,
        edge_perpendicular)]

    jobs1 = [ct, on] + [(None, l2_edges[i][1]) for i in MEAN_IDX]
    s0_ct, s0_on, cnt_nv, cnt_nh, cnt_pp = _sc_scatter(16, x_pad, jobs1)

    h = _tc_layer0(x_pad, s0_ct, s0_on, _pad_w(W0_root),
                   _pad_w(W0_connected_to), _pad_w(W0_ordered_next))

    s1_ct, s1_on = _sc_scatter(16, h, [ct, on])

    h1 = _tc_layer1(h, s1_ct, s1_on, W1_root, W1_skip,
                    W1_connected_to, W1_ordered_next)

    s2 = _sc_scatter(32, h1, l2_edges)

    h2 = _tc_layer2(h1, s2, [cnt_nv, cnt_nh, cnt_pp], W2_root, W2_skip,
                    [W2_represents, W2_represented_by,
                     W2_neighboring_vertical, W2_neighboring_horizontal,
                     W2_contains, W2_order, W2_perpendicular])

    return h2[:N]


# speedup vs baseline: 2.5118x; 1.0017x over previous
import functools

import jax
import jax.numpy as jnp
from jax import lax
from jax.experimental import pallas as pl
from jax.experimental.pallas import tpu as pltpu
from jax.experimental.pallas import tpu_sc as plsc

N = 50000
E = 100000

NC = 2
NS = 16
L = 16

TPT = 3136
NPAD = NS * TPT
G = 128
GPT = 56
NG = NS * GPT
EPAD = NG * G


def _sc_scatter(feat, table, zeros, jobs):
    njobs = len(jobs)
    rounds = []
    for r in range((njobs + 1) // 2):
        c0 = (2 * r, jobs[2 * r])
        c1 = (2 * r + 1, jobs[2 * r + 1]) if 2 * r + 1 < njobs else None
        rounds.append((c0, c1))

    idx_inputs = []
    job_idx_pos = []
    for src, dst in jobs:
        if src is None:
            job_idx_pos.append((None, len(idx_inputs)))
            idx_inputs.append(dst)
        else:
            job_idx_pos.append((len(idx_inputs), len(idx_inputs) + 1))
            idx_inputs.append(src)
            idx_inputs.append(dst)

    n_idx = len(idx_inputs)
    mesh = plsc.VectorSubcoreMesh(core_axis_name="c", subcore_axis_name="s")

    @functools.partial(
        pl.kernel,
        out_type=[jax.ShapeDtypeStruct((NPAD, feat), jnp.float32)] * njobs,
        mesh=mesh,
        scratch_types=[
            pltpu.MemorySpace.VMEM_SHARED((NPAD, feat), jnp.float32),
            pltpu.MemorySpace.VMEM((GPT, G), jnp.int32),
            pltpu.MemorySpace.VMEM((GPT, G), jnp.int32),
            pltpu.MemorySpace.VMEM((2, G, feat), jnp.float32),
            pltpu.SemaphoreType.DMA,
            pltpu.SemaphoreType.DMA,
        ],
        compiler_params=pltpu.CompilerParams(use_tc_tiling_on_sc=False),
    )
    def run(*refs):
        table_ref = refs[0]
        zeros_ref = refs[1]
        idx_refs = refs[2:2 + n_idx]
        out_refs = refs[2 + n_idx:2 + n_idx + njobs]
        acc, sidx, didx, rows, sem0, sem1 = refs[2 + n_idx + njobs:]

        c = lax.axis_index("c")
        s = lax.axis_index("s")
        base = s * TPT
        gbase = s * GPT

        one16 = jnp.ones((L,), jnp.float32)

        def emit_job(job_id):
            src_pos, dst_pos = job_idx_pos[job_id]
            out_ref = out_refs[job_id]

            def zero_acc():
                pltpu.sync_copy(zeros_ref, acc.at[pl.ds(base, TPT)])

            def accumulate():
                pltpu.sync_copy(idx_refs[dst_pos].at[pl.ds(gbase, GPT)],
                                didx)
                if src_pos is not None:
                    pltpu.sync_copy(idx_refs[src_pos].at[pl.ds(gbase, GPT)],
                                    sidx)
                    pltpu.async_copy(table_ref.at[sidx.at[0]], rows.at[0],
                                     sem0)

                    def grp(t, carry):
                        g0 = 2 * t
                        pltpu.async_copy(table_ref.at[sidx.at[g0 + 1]],
                                         rows.at[1], sem1)
                        pltpu.make_async_copy(
                            table_ref.at[sidx.at[g0]], rows.at[0],
                            sem0).wait()
                        pltpu.sync_copy(rows.at[0], acc.at[didx.at[g0]],
                                        add=True)

                        @pl.when(g0 + 2 < GPT)
                        def _():
                            pltpu.async_copy(table_ref.at[sidx.at[g0 + 2]],
                                             rows.at[0], sem0)

                        pltpu.make_async_copy(
                            table_ref.at[sidx.at[g0 + 1]], rows.at[1],
                            sem1).wait()
                        pltpu.sync_copy(rows.at[1], acc.at[didx.at[g0 + 1]],
                                        add=True)
                        return carry

                    lax.fori_loop(0, GPT // 2, grp, 0)
                else:
                    def fill_ones(i, carry):
                        for k in range(feat // L):
                            rows[0, i, pl.ds(k * L, L)] = one16
                        return carry

                    lax.fori_loop(0, G, fill_ones, 0)

                    def grp(g, carry):
                        pltpu.sync_copy(rows.at[0], acc.at[didx.at[g]],
                                        add=True)
                        return carry

                    lax.fori_loop(0, GPT, grp, 0)

            def flush():
                pltpu.sync_copy(acc.at[pl.ds(base, TPT)],
                                out_ref.at[pl.ds(base, TPT)])

            return zero_acc, accumulate, flush

        for c0_job, c1_job in rounds:
            stages0 = emit_job(c0_job[0])
            stages1 = emit_job(c1_job[0]) if c1_job is not None else None
            pl.when(c == 0)(stages0[0])
            if stages1 is not None:
                pl.when(c == 1)(stages1[0])
            plsc.subcore_barrier()
            pl.when(c == 0)(stages0[1])
            if stages1 is not None:
                pl.when(c == 1)(stages1[1])
            plsc.subcore_barrier()
            pl.when(c == 0)(stages0[2])
            if stages1 is not None:
                pl.when(c == 1)(stages1[2])
            plsc.subcore_barrier()

    return run(table, zeros, *idx_inputs)


BLK = 1024
GRID = NPAD // BLK


def _row_spec(f):
    return pl.BlockSpec((BLK, f), lambda i: (i, 0))


def _full_spec(fi, fo):
    return pl.BlockSpec((fi, fo), lambda i: (0, 0))


def _tc_layer0(x, s_ct, s_on, w_r, w_ct, w_on):
    def body(x_ref, a_ref, b_ref, wr_ref, wa_ref, wb_ref, o_ref):
        acc = jnp.dot(x_ref[...], wr_ref[...], preferred_element_type=jnp.float32)
        acc += jnp.dot(a_ref[...], wa_ref[...], preferred_element_type=jnp.float32)
        acc += jnp.dot(b_ref[...], wb_ref[...], preferred_element_type=jnp.float32)
        o_ref[...] = acc

    return pl.pallas_call(
        body,
        grid=(GRID,),
        in_specs=[_row_spec(16), _row_spec(16), _row_spec(16),
                  _full_spec(16, 16), _full_spec(16, 16), _full_spec(16, 16)],
        out_specs=_row_spec(16),
        out_shape=jax.ShapeDtypeStruct((NPAD, 16), jnp.float32),
    )(x, s_ct, s_on, w_r, w_ct, w_on)


def _tc_layer1(h, s_ct, s_on, w_r, w_skip, w_ct, w_on):
    def body(h_ref, a_ref, b_ref, wr_ref, ws_ref, wa_ref, wb_ref, o_ref):
        acc = jnp.dot(h_ref[...], wr_ref[...] + ws_ref[...],
                      preferred_element_type=jnp.float32)
        acc += jnp.dot(a_ref[...], wa_ref[...], preferred_element_type=jnp.float32)
        acc += jnp.dot(b_ref[...], wb_ref[...], preferred_element_type=jnp.float32)
        o_ref[...] = acc

    return pl.pallas_call(
        body,
        grid=(GRID,),
        in_specs=[_row_spec(16), _row_spec(16), _row_spec(16),
                  _full_spec(16, 32), _full_spec(16, 32),
                  _full_spec(16, 32), _full_spec(16, 32)],
        out_specs=_row_spec(32),
        out_shape=jax.ShapeDtypeStruct((NPAD, 32), jnp.float32),
    )(h, s_ct, s_on, w_r, w_skip, w_ct, w_on)


MEAN_IDX = (2, 3, 6)


def _tc_layer2(h1, s_list, cnt_list, w_r, w_skip, w_list):
    def body(*refs):
        h_ref = refs[0]
        s_refs = refs[1:8]
        c_refs = refs[8:11]
        wr_ref, ws_ref = refs[11], refs[12]
        w_refs = refs[13:20]
        o_ref = refs[20]
        acc = jnp.dot(h_ref[...], wr_ref[...] + ws_ref[...],
                      preferred_element_type=jnp.float32)
        ci = 0
        for i in range(7):
            m = s_refs[i][...]
            if i in MEAN_IDX:
                cnt = c_refs[ci][...][:, 0:1]
                m = m / jnp.maximum(cnt, 1.0)
                ci += 1
            acc += jnp.dot(m, w_refs[i][...],
                           preferred_element_type=jnp.float32)
        o_ref[...] = jnp.maximum(acc, 0.0)

    return pl.pallas_call(
        body,
        grid=(GRID,),
        in_specs=([_row_spec(32)] + [_row_spec(32)] * 7 + [_row_spec(16)] * 3
                  + [_full_spec(32, 64)] * 9),
        out_specs=_row_spec(64),
        out_shape=jax.ShapeDtypeStruct((NPAD, 64), jnp.float32),
    )(h1, *s_list, *cnt_list, w_r, w_skip, *w_list)


def _prep_edges(ei):
    pad = jnp.full((EPAD - E,), N, jnp.int32)
    src = jnp.concatenate([ei[0], pad]).reshape(NG, G)
    dst = jnp.concatenate([ei[1], pad]).reshape(NG, G)
    return src, dst


def _pad_w(w):
    return jnp.zeros((16, w.shape[1]), jnp.float32).at[:w.shape[0]].set(w)


def kernel(x, edge_connected_to, edge_ordered_next, edge_represents,
           edge_represented_by, edge_neighboring_vertical,
           edge_neighboring_horizontal, edge_contains, edge_order,
           edge_perpendicular, W0_connected_to, W0_ordered_next, W0_root,
           W1_connected_to, W1_ordered_next, W1_root, W1_skip,
           W2_represents, W2_represented_by, W2_neighboring_vertical,
           W2_neighboring_horizontal, W2_contains, W2_order,
           W2_perpendicular, W2_root, W2_skip):
    x_pad = jnp.zeros((NPAD, 16), jnp.float32).at[:N, :6].set(x)

    ct = _prep_edges(edge_connected_to)
    on = _prep_edges(edge_ordered_next)
    l2_edges = [_prep_edges(e) for e in (
        edge_represents, edge_represented_by, edge_neighboring_vertical,
        edge_neighboring_horizontal, edge_contains, edge_order,
        edge_perpendicular)]

    z16 = jnp.zeros((TPT, 16), jnp.float32)
    z32 = jnp.zeros((TPT, 32), jnp.float32)

    jobs1 = [ct, on] + [(None, l2_edges[i][1]) for i in MEAN_IDX]
    s0_ct, s0_on, cnt_nv, cnt_nh, cnt_pp = _sc_scatter(16, x_pad, z16, jobs1)

    h = _tc_layer0(x_pad, s0_ct, s0_on, _pad_w(W0_root),
                   _pad_w(W0_connected_to), _pad_w(W0_ordered_next))

    s1_ct, s1_on = _sc_scatter(16, h, z16, [ct, on])

    h1 = _tc_layer1(h, s1_ct, s1_on, W1_root, W1_skip,
                    W1_connected_to, W1_ordered_next)

    s2 = _sc_scatter(32, h1, z32, l2_edges)

    h2 = _tc_layer2(h1, s2, [cnt_nv, cnt_nh, cnt_pp], W2_root, W2_skip,
                    [W2_represents, W2_represented_by,
                     W2_neighboring_vertical, W2_neighboring_horizontal,
                     W2_contains, W2_order, W2_perpendicular])

    return h2[:N]


# speedup vs baseline: 6.3111x; 2.5126x over previous
import functools

import jax
import jax.numpy as jnp
from jax import lax
from jax.experimental import pallas as pl
from jax.experimental.pallas import tpu as pltpu
from jax.experimental.pallas import tpu_sc as plsc

N = 50000
E = 100000

NC = 2
NS = 16
L = 16

TPT = 3136
NPAD = NS * TPT

G = 128
GPT_FULL = 54
GPT_HALF = 27


def _sc_scatter(feat, table, zeros, jobs):
    njobs = len(jobs)
    rounds = []
    for r in range((njobs + 1) // 2):
        c0 = 2 * r
        c1 = 2 * r + 1 if 2 * r + 1 < njobs else None
        rounds.append((c0, c1))

    idx_inputs = []
    job_meta = []
    for src, dst, gpt in jobs:
        assert gpt % 3 == 0
        if src is None:
            job_meta.append((None, len(idx_inputs), gpt))
            idx_inputs.append(dst)
        else:
            job_meta.append((len(idx_inputs), len(idx_inputs) + 1, gpt))
            idx_inputs.append(src)
            idx_inputs.append(dst)

    n_idx = len(idx_inputs)
    max_gpt = max(m[2] for m in job_meta)
    mesh = plsc.VectorSubcoreMesh(core_axis_name="c", subcore_axis_name="s")

    @functools.partial(
        pl.kernel,
        out_type=[jax.ShapeDtypeStruct((NPAD, feat), jnp.float32)] * njobs,
        mesh=mesh,
        scratch_types=[
            pltpu.MemorySpace.VMEM_SHARED((NPAD, feat), jnp.float32),
            pltpu.MemorySpace.VMEM((max_gpt, G), jnp.int32),
            pltpu.MemorySpace.VMEM((max_gpt, G), jnp.int32),
            pltpu.MemorySpace.VMEM((3, G, feat), jnp.float32),
            pltpu.SemaphoreType.DMA,
            pltpu.SemaphoreType.DMA,
            pltpu.SemaphoreType.DMA,
            pltpu.SemaphoreType.DMA,
            pltpu.SemaphoreType.DMA,
            pltpu.SemaphoreType.DMA,
        ],
        compiler_params=pltpu.CompilerParams(use_tc_tiling_on_sc=False),
    )
    def run(*refs):
        table_ref = refs[0]
        zeros_ref = refs[1]
        idx_refs = refs[2:2 + n_idx]
        out_refs = refs[2 + n_idx:2 + n_idx + njobs]
        (acc, sidx, didx, rows,
         gs0, gs1, gs2, ss0, ss1, ss2) = refs[2 + n_idx + njobs:]
        gsem = (gs0, gs1, gs2)
        ssem = (ss0, ss1, ss2)

        c = lax.axis_index("c")
        s = lax.axis_index("s")
        base = s * TPT

        one16 = jnp.ones((L,), jnp.float32)

        def emit_job(job_id):
            src_pos, dst_pos, gpt = job_meta[job_id]
            out_ref = out_refs[job_id]
            gbase = s * gpt

            def zero_acc():
                pltpu.sync_copy(zeros_ref, acc.at[pl.ds(base, TPT)])

            def issue_gather(g, b):
                pltpu.async_copy(table_ref.at[sidx.at[g]], rows.at[b],
                                 gsem[b])

            def wait_gather(g, b):
                pltpu.make_async_copy(table_ref.at[sidx.at[g]], rows.at[b],
                                      gsem[b]).wait()

            def issue_scatter(g, b):
                pltpu.async_copy(rows.at[b], acc.at[didx.at[g]], ssem[b],
                                 add=True)

            def wait_scatter(g, b):
                pltpu.make_async_copy(rows.at[b], acc.at[didx.at[g]],
                                      ssem[b]).wait()

            def accumulate():
                pltpu.sync_copy(idx_refs[dst_pos].at[pl.ds(gbase, gpt)],
                                didx.at[pl.ds(0, gpt)])
                if src_pos is not None:
                    pltpu.sync_copy(idx_refs[src_pos].at[pl.ds(gbase, gpt)],
                                    sidx.at[pl.ds(0, gpt)])
                    issue_gather(0, 0)
                    issue_gather(1, 1)

                    def pipe(t, carry):
                        for j in range(3):
                            g = 3 * t + j

                            @pl.when(g + 2 < gpt)
                            def _():
                                @pl.when(g >= 1)
                                def _():
                                    wait_scatter(g - 1, (j + 2) % 3)

                                issue_gather(g + 2, (j + 2) % 3)

                            wait_gather(g, j)
                            issue_scatter(g, j)
                        return carry

                    lax.fori_loop(0, gpt // 3, pipe, 0)
                    wait_scatter(gpt - 3, 0)
                    wait_scatter(gpt - 2, 1)
                    wait_scatter(gpt - 1, 2)
                else:
                    def fill_ones(i, carry):
                        for k in range(feat // L):
                            rows[0, i, pl.ds(k * L, L)] = one16
                        return carry

                    lax.fori_loop(0, G, fill_ones, 0)

                    def pipe(t, carry):
                        for j in range(3):
                            g = 3 * t + j

                            @pl.when(g >= 3)
                            def _():
                                pltpu.make_async_copy(
                                    rows.at[0], acc.at[didx.at[g - 3]],
                                    ssem[j]).wait()

                            pltpu.async_copy(rows.at[0], acc.at[didx.at[g]],
                                             ssem[j], add=True)
                        return carry

                    lax.fori_loop(0, gpt // 3, pipe, 0)
                    wait_scatter(gpt - 3, 0)
                    wait_scatter(gpt - 2, 1)
                    wait_scatter(gpt - 1, 2)

            def flush():
                pltpu.sync_copy(acc.at[pl.ds(base, TPT)],
                                out_ref.at[pl.ds(base, TPT)])

            return zero_acc, accumulate, flush

        for c0_job, c1_job in rounds:
            stages0 = emit_job(c0_job)
            stages1 = emit_job(c1_job) if c1_job is not None else None
            pl.when(c == 0)(stages0[0])
            if stages1 is not None:
                pl.when(c == 1)(stages1[0])
            plsc.subcore_barrier()
            pl.when(c == 0)(stages0[1])
            if stages1 is not None:
                pl.when(c == 1)(stages1[1])
            plsc.subcore_barrier()
            pl.when(c == 0)(stages0[2])
            if stages1 is not None:
                pl.when(c == 1)(stages1[2])
            plsc.subcore_barrier()

    return run(table, zeros, *idx_inputs)


BLK = 1024
GRID = NPAD // BLK


def _row_spec(f):
    return pl.BlockSpec((BLK, f), lambda i: (i, 0))


def _full_spec(fi, fo):
    return pl.BlockSpec((fi, fo), lambda i: (0, 0))


def _tc_layer0(x, s_ct, s_on, w_r, w_ct, w_on):
    def body(x_ref, a_ref, b_ref, wr_ref, wa_ref, wb_ref, o_ref):
        acc = jnp.dot(x_ref[...], wr_ref[...], preferred_element_type=jnp.float32)
        acc += jnp.dot(a_ref[...], wa_ref[...], preferred_element_type=jnp.float32)
        acc += jnp.dot(b_ref[...], wb_ref[...], preferred_element_type=jnp.float32)
        o_ref[...] = acc

    return pl.pallas_call(
        body,
        grid=(GRID,),
        in_specs=[_row_spec(16), _row_spec(16), _row_spec(16),
                  _full_spec(16, 16), _full_spec(16, 16), _full_spec(16, 16)],
        out_specs=_row_spec(16),
        out_shape=jax.ShapeDtypeStruct((NPAD, 16), jnp.float32),
    )(x, s_ct, s_on, w_r, w_ct, w_on)


def _tc_layer1(h, s_ct, s_on, w_r, w_skip, w_ct, w_on):
    def body(h_ref, a_ref, b_ref, wr_ref, ws_ref, wa_ref, wb_ref, o_ref):
        acc = jnp.dot(h_ref[...], wr_ref[...] + ws_ref[...],
                      preferred_element_type=jnp.float32)
        acc += jnp.dot(a_ref[...], wa_ref[...], preferred_element_type=jnp.float32)
        acc += jnp.dot(b_ref[...], wb_ref[...], preferred_element_type=jnp.float32)
        o_ref[...] = acc

    return pl.pallas_call(
        body,
        grid=(GRID,),
        in_specs=[_row_spec(16), _row_spec(16), _row_spec(16),
                  _full_spec(16, 32), _full_spec(16, 32),
                  _full_spec(16, 32), _full_spec(16, 32)],
        out_specs=_row_spec(32),
        out_shape=jax.ShapeDtypeStruct((NPAD, 32), jnp.float32),
    )(h, s_ct, s_on, w_r, w_skip, w_ct, w_on)


MEAN_IDX = (2, 3, 6)


def _tc_layer2(h1, s_list, cnt_list, w_r, w_skip, w_list):
    def body(*refs):
        h_ref = refs[0]
        s_refs = refs[1:9]
        c_refs = refs[9:12]
        wr_ref, ws_ref = refs[12], refs[13]
        w_refs = refs[14:21]
        o_ref = refs[21]
        acc = jnp.dot(h_ref[...], wr_ref[...] + ws_ref[...],
                      preferred_element_type=jnp.float32)
        acc += jnp.dot(s_refs[0][...] + s_refs[1][...], w_refs[0][...],
                       preferred_element_type=jnp.float32)
        ci = 0
        for i in range(1, 7):
            m = s_refs[i + 1][...]
            if i in MEAN_IDX:
                cnt = c_refs[ci][...][:, 0:1]
                m = m / jnp.maximum(cnt, 1.0)
                ci += 1
            acc += jnp.dot(m, w_refs[i][...],
                           preferred_element_type=jnp.float32)
        o_ref[...] = jnp.maximum(acc, 0.0)

    return pl.pallas_call(
        body,
        grid=(GRID,),
        in_specs=([_row_spec(32)] + [_row_spec(32)] * 8 + [_row_spec(16)] * 3
                  + [_full_spec(32, 64)] * 9),
        out_specs=_row_spec(64),
        out_shape=jax.ShapeDtypeStruct((NPAD, 64), jnp.float32),
    )(h1, *s_list, *cnt_list, w_r, w_skip, *w_list)


def _pad_idx(idx, gpt):
    npad = NS * gpt * G - idx.shape[0]
    fill = N + jnp.arange(npad, dtype=jnp.int32) % (NPAD - N)
    return jnp.concatenate([idx, fill]).reshape(NS * gpt, G)


def _prep_edges(ei, gpt=GPT_FULL):
    return _pad_idx(ei[0], gpt), _pad_idx(ei[1], gpt), gpt


def _pad_w(w):
    return jnp.zeros((16, w.shape[1]), jnp.float32).at[:w.shape[0]].set(w)


def kernel(x, edge_connected_to, edge_ordered_next, edge_represents,
           edge_represented_by, edge_neighboring_vertical,
           edge_neighboring_horizontal, edge_contains, edge_order,
           edge_perpendicular, W0_connected_to, W0_ordered_next, W0_root,
           W1_connected_to, W1_ordered_next, W1_root, W1_skip,
           W2_represents, W2_represented_by, W2_neighboring_vertical,
           W2_neighboring_horizontal, W2_contains, W2_order,
           W2_perpendicular, W2_root, W2_skip):
    x_pad = jnp.zeros((NPAD, 16), jnp.float32).at[:N, :6].set(x)

    ct = _prep_edges(edge_connected_to)
    on = _prep_edges(edge_ordered_next)
    l2 = [edge_represents, edge_represented_by, edge_neighboring_vertical,
          edge_neighboring_horizontal, edge_contains, edge_order,
          edge_perpendicular]
    r0a = (_pad_idx(l2[0][0, :E // 2], GPT_HALF),
           _pad_idx(l2[0][1, :E // 2], GPT_HALF), GPT_HALF)
    r0b = (_pad_idx(l2[0][0, E // 2:], GPT_HALF),
           _pad_idx(l2[0][1, E // 2:], GPT_HALF), GPT_HALF)
    l2_full = [_prep_edges(e) for e in l2[1:]]

    z16 = jnp.zeros((TPT, 16), jnp.float32)
    z32 = jnp.zeros((TPT, 32), jnp.float32)

    cnt_jobs = [(None, _pad_idx(l2[i][1], GPT_FULL), GPT_FULL)
                for i in MEAN_IDX]
    jobs1 = [ct, on] + cnt_jobs
    s0_ct, s0_on, cnt_nv, cnt_nh, cnt_pp = _sc_scatter(16, x_pad, z16, jobs1)

    h = _tc_layer0(x_pad, s0_ct, s0_on, _pad_w(W0_root),
                   _pad_w(W0_connected_to), _pad_w(W0_ordered_next))

    s1_ct, s1_on = _sc_scatter(16, h, z16, [ct, on])

    h1 = _tc_layer1(h, s1_ct, s1_on, W1_root, W1_skip,
                    W1_connected_to, W1_ordered_next)

    jobs3 = [r0a, r0b, l2_full[1], l2_full[0], l2_full[3], l2_full[2],
             l2_full[5], l2_full[4]]
    s2 = _sc_scatter(32, h1, z32, jobs3)
    s_list = [s2[0], s2[1],
              s2[3], s2[2], s2[5], s2[4], s2[7], s2[6]]

    h2 = _tc_layer2(h1, s_list, [cnt_nv, cnt_nh, cnt_pp], W2_root, W2_skip,
                    [W2_represents, W2_represented_by,
                     W2_neighboring_vertical, W2_neighboring_horizontal,
                     W2_contains, W2_order, W2_perpendicular])

    return h2[:N]


# speedup vs baseline: 6.3917x; 1.0128x over previous
import functools

import jax
import jax.numpy as jnp
from jax import lax
from jax.experimental import pallas as pl
from jax.experimental.pallas import tpu as pltpu
from jax.experimental.pallas import tpu_sc as plsc

N = 50000
E = 100000

NC = 2
NS = 16
L = 16

TPT = 3136
NPAD = NS * TPT

G = 128
GPT_FULL = 51
GPT_HALF = 27


def _sc_scatter(feat, table, zeros, jobs, table_in_spmem=False):
    njobs = len(jobs)
    rounds = []
    for r in range((njobs + 1) // 2):
        c0 = 2 * r
        c1 = 2 * r + 1 if 2 * r + 1 < njobs else None
        rounds.append((c0, c1))

    idx_inputs = []
    job_meta = []
    for src, dst, gpt in jobs:
        assert gpt % 3 == 0
        if src is None:
            job_meta.append((None, len(idx_inputs), gpt))
            idx_inputs.append(dst)
        else:
            job_meta.append((len(idx_inputs), len(idx_inputs) + 1, gpt))
            idx_inputs.append(src)
            idx_inputs.append(dst)

    n_idx = len(idx_inputs)
    max_gpt = max(m[2] for m in job_meta)
    mesh = plsc.VectorSubcoreMesh(core_axis_name="c", subcore_axis_name="s")

    tbl_scratch = ([pltpu.MemorySpace.VMEM_SHARED((NPAD, feat), jnp.float32)]
                   if table_in_spmem else [])

    @functools.partial(
        pl.kernel,
        out_type=[jax.ShapeDtypeStruct((NPAD, feat), jnp.float32)] * njobs,
        mesh=mesh,
        scratch_types=tbl_scratch + [
            pltpu.MemorySpace.VMEM_SHARED((NPAD, feat), jnp.float32),
            pltpu.MemorySpace.VMEM((max_gpt, G), jnp.int32),
            pltpu.MemorySpace.VMEM((max_gpt, G), jnp.int32),
            pltpu.MemorySpace.VMEM((3, G, feat), jnp.float32),
            pltpu.SemaphoreType.DMA,
            pltpu.SemaphoreType.DMA,
            pltpu.SemaphoreType.DMA,
            pltpu.SemaphoreType.DMA,
            pltpu.SemaphoreType.DMA,
            pltpu.SemaphoreType.DMA,
        ],
        compiler_params=pltpu.CompilerParams(use_tc_tiling_on_sc=False),
    )
    def run(*refs):
        table_hbm = refs[0]
        zeros_ref = refs[1]
        idx_refs = refs[2:2 + n_idx]
        out_refs = refs[2 + n_idx:2 + n_idx + njobs]
        scratch = refs[2 + n_idx + njobs:]
        if table_in_spmem:
            tbl, scratch = scratch[0], scratch[1:]
        (acc, sidx, didx, rows,
         gs0, gs1, gs2, ss0, ss1, ss2) = scratch
        gsem = (gs0, gs1, gs2)
        ssem = (ss0, ss1, ss2)

        c = lax.axis_index("c")
        s = lax.axis_index("s")
        base = s * TPT

        if table_in_spmem:
            pltpu.sync_copy(table_hbm.at[pl.ds(base, TPT)],
                            tbl.at[pl.ds(base, TPT)])
            plsc.subcore_barrier()
            table_ref = tbl
        else:
            table_ref = table_hbm

        one16 = jnp.ones((L,), jnp.float32)

        def emit_job(job_id):
            src_pos, dst_pos, gpt = job_meta[job_id]
            out_ref = out_refs[job_id]
            gbase = s * gpt

            def zero_acc():
                pltpu.sync_copy(zeros_ref, acc.at[pl.ds(base, TPT)])

            def issue_gather(g, b):
                pltpu.async_copy(table_ref.at[sidx.at[g]], rows.at[b],
                                 gsem[b])

            def wait_gather(g, b):
                pltpu.make_async_copy(table_ref.at[sidx.at[g]], rows.at[b],
                                      gsem[b]).wait()

            def issue_scatter(g, b):
                pltpu.async_copy(rows.at[b], acc.at[didx.at[g]], ssem[b],
                                 add=True)

            def wait_scatter(g, b):
                pltpu.make_async_copy(rows.at[b], acc.at[didx.at[g]],
                                      ssem[b]).wait()

            def accumulate():
                pltpu.sync_copy(idx_refs[dst_pos].at[pl.ds(gbase, gpt)],
                                didx.at[pl.ds(0, gpt)])
                if src_pos is not None:
                    pltpu.sync_copy(idx_refs[src_pos].at[pl.ds(gbase, gpt)],
                                    sidx.at[pl.ds(0, gpt)])
                    issue_gather(0, 0)
                    issue_gather(1, 1)

                    def pipe(t, carry):
                        for j in range(3):
                            g = 3 * t + j

                            @pl.when(g + 2 < gpt)
                            def _():
                                @pl.when(g >= 1)
                                def _():
                                    wait_scatter(g - 1, (j + 2) % 3)

                                issue_gather(g + 2, (j + 2) % 3)

                            wait_gather(g, j)
                            issue_scatter(g, j)
                        return carry

                    lax.fori_loop(0, gpt // 3, pipe, 0)
                    wait_scatter(gpt - 3, 0)
                    wait_scatter(gpt - 2, 1)
                    wait_scatter(gpt - 1, 2)
                else:
                    def fill_ones(i, carry):
                        for k in range(feat // L):
                            rows[0, i, pl.ds(k * L, L)] = one16
                        return carry

                    lax.fori_loop(0, G, fill_ones, 0)

                    def pipe(t, carry):
                        for j in range(3):
                            g = 3 * t + j

                            @pl.when(g >= 3)
                            def _():
                                pltpu.make_async_copy(
                                    rows.at[0], acc.at[didx.at[g - 3]],
                                    ssem[j]).wait()

                            pltpu.async_copy(rows.at[0], acc.at[didx.at[g]],
                                             ssem[j], add=True)
                        return carry

                    lax.fori_loop(0, gpt // 3, pipe, 0)
                    wait_scatter(gpt - 3, 0)
                    wait_scatter(gpt - 2, 1)
                    wait_scatter(gpt - 1, 2)

            def flush():
                pltpu.sync_copy(acc.at[pl.ds(base, TPT)],
                                out_ref.at[pl.ds(base, TPT)])

            return zero_acc, accumulate, flush

        for c0_job, c1_job in rounds:
            stages0 = emit_job(c0_job)
            stages1 = emit_job(c1_job) if c1_job is not None else None
            pl.when(c == 0)(stages0[0])
            if stages1 is not None:
                pl.when(c == 1)(stages1[0])
            plsc.subcore_barrier()
            pl.when(c == 0)(stages0[1])
            if stages1 is not None:
                pl.when(c == 1)(stages1[1])
            plsc.subcore_barrier()
            pl.when(c == 0)(stages0[2])
            if stages1 is not None:
                pl.when(c == 1)(stages1[2])
            plsc.subcore_barrier()

    return run(table, zeros, *idx_inputs)


BLK = 1024
GRID = NPAD // BLK


def _row_spec(f):
    return pl.BlockSpec((BLK, f), lambda i: (i, 0))


def _full_spec(fi, fo):
    return pl.BlockSpec((fi, fo), lambda i: (0, 0))


def _tc_layer0(x, s_ct, s_on, w_r, w_ct, w_on):
    def body(x_ref, a_ref, b_ref, wr_ref, wa_ref, wb_ref, o_ref):
        acc = jnp.dot(x_ref[...], wr_ref[...], preferred_element_type=jnp.float32)
        acc += jnp.dot(a_ref[...], wa_ref[...], preferred_element_type=jnp.float32)
        acc += jnp.dot(b_ref[...], wb_ref[...], preferred_element_type=jnp.float32)
        o_ref[...] = acc

    return pl.pallas_call(
        body,
        grid=(GRID,),
        in_specs=[_row_spec(16), _row_spec(16), _row_spec(16),
                  _full_spec(16, 16), _full_spec(16, 16), _full_spec(16, 16)],
        out_specs=_row_spec(16),
        out_shape=jax.ShapeDtypeStruct((NPAD, 16), jnp.float32),
    )(x, s_ct, s_on, w_r, w_ct, w_on)


def _tc_layer1(h, s_ct, s_on, w_r, w_skip, w_ct, w_on):
    def body(h_ref, a_ref, b_ref, wr_ref, ws_ref, wa_ref, wb_ref, o_ref):
        acc = jnp.dot(h_ref[...], wr_ref[...] + ws_ref[...],
                      preferred_element_type=jnp.float32)
        acc += jnp.dot(a_ref[...], wa_ref[...], preferred_element_type=jnp.float32)
        acc += jnp.dot(b_ref[...], wb_ref[...], preferred_element_type=jnp.float32)
        o_ref[...] = acc

    return pl.pallas_call(
        body,
        grid=(GRID,),
        in_specs=[_row_spec(16), _row_spec(16), _row_spec(16),
                  _full_spec(16, 32), _full_spec(16, 32),
                  _full_spec(16, 32), _full_spec(16, 32)],
        out_specs=_row_spec(32),
        out_shape=jax.ShapeDtypeStruct((NPAD, 32), jnp.float32),
    )(h, s_ct, s_on, w_r, w_skip, w_ct, w_on)


MEAN_IDX = (2, 3, 6)


def _tc_layer2(h1, s_list, cnt_list, w_r, w_skip, w_list):
    def body(*refs):
        h_ref = refs[0]
        s_refs = refs[1:9]
        c_refs = refs[9:12]
        wr_ref, ws_ref = refs[12], refs[13]
        w_refs = refs[14:21]
        o_ref = refs[21]
        acc = jnp.dot(h_ref[...], wr_ref[...] + ws_ref[...],
                      preferred_element_type=jnp.float32)
        acc += jnp.dot(s_refs[0][...] + s_refs[1][...], w_refs[0][...],
                       preferred_element_type=jnp.float32)
        ci = 0
        for i in range(1, 7):
            m = s_refs[i + 1][...]
            if i in MEAN_IDX:
                cnt = c_refs[ci][...][:, 0:1]
                m = m / jnp.maximum(cnt, 1.0)
                ci += 1
            acc += jnp.dot(m, w_refs[i][...],
                           preferred_element_type=jnp.float32)
        o_ref[...] = jnp.maximum(acc, 0.0)

    return pl.pallas_call(
        body,
        grid=(GRID,),
        in_specs=([_row_spec(32)] + [_row_spec(32)] * 8 + [_row_spec(16)] * 3
                  + [_full_spec(32, 64)] * 9),
        out_specs=_row_spec(64),
        out_shape=jax.ShapeDtypeStruct((NPAD, 64), jnp.float32),
    )(h1, *s_list, *cnt_list, w_r, w_skip, *w_list)


def _pad_idx(idx, gpt):
    npad = NS * gpt * G - idx.shape[0]
    fill = N + jnp.arange(npad, dtype=jnp.int32) % (NPAD - N)
    return jnp.concatenate([idx, fill]).reshape(NS * gpt, G)


def _prep_edges(ei, gpt=GPT_FULL):
    return _pad_idx(ei[0], gpt), _pad_idx(ei[1], gpt), gpt


def _pad_w(w):
    return jnp.zeros((16, w.shape[1]), jnp.float32).at[:w.shape[0]].set(w)


def kernel(x, edge_connected_to, edge_ordered_next, edge_represents,
           edge_represented_by, edge_neighboring_vertical,
           edge_neighboring_horizontal, edge_contains, edge_order,
           edge_perpendicular, W0_connected_to, W0_ordered_next, W0_root,
           W1_connected_to, W1_ordered_next, W1_root, W1_skip,
           W2_represents, W2_represented_by, W2_neighboring_vertical,
           W2_neighboring_horizontal, W2_contains, W2_order,
           W2_perpendicular, W2_root, W2_skip):
    x_pad = jnp.zeros((NPAD, 16), jnp.float32).at[:N, :6].set(x)

    ct = _prep_edges(edge_connected_to)
    on = _prep_edges(edge_ordered_next)
    l2 = [edge_represents, edge_represented_by, edge_neighboring_vertical,
          edge_neighboring_horizontal, edge_contains, edge_order,
          edge_perpendicular]
    r0a = (_pad_idx(l2[0][0, :E // 2], GPT_HALF),
           _pad_idx(l2[0][1, :E // 2], GPT_HALF), GPT_HALF)
    r0b = (_pad_idx(l2[0][0, E // 2:], GPT_HALF),
           _pad_idx(l2[0][1, E // 2:], GPT_HALF), GPT_HALF)
    l2_full = [_prep_edges(e) for e in l2[1:]]

    z16 = jnp.zeros((TPT, 16), jnp.float32)
    z32 = jnp.zeros((TPT, 32), jnp.float32)

    cnt_jobs = [(None, _pad_idx(l2[i][1], GPT_FULL), GPT_FULL)
                for i in MEAN_IDX]
    jobs1 = [ct, on] + cnt_jobs
    s0_ct, s0_on, cnt_nv, cnt_nh, cnt_pp = _sc_scatter(
        16, x_pad, z16, jobs1, table_in_spmem=True)

    h = _tc_layer0(x_pad, s0_ct, s0_on, _pad_w(W0_root),
                   _pad_w(W0_connected_to), _pad_w(W0_ordered_next))

    s1_ct, s1_on = _sc_scatter(16, h, z16, [ct, on], table_in_spmem=True)

    h1 = _tc_layer1(h, s1_ct, s1_on, W1_root, W1_skip,
                    W1_connected_to, W1_ordered_next)

    jobs3 = [r0a, r0b, l2_full[1], l2_full[0], l2_full[3], l2_full[2],
             l2_full[5], l2_full[4]]
    s2 = _sc_scatter(32, h1, z32, jobs3)
    s_list = [s2[0], s2[1],
              s2[3], s2[2], s2[5], s2[4], s2[7], s2[6]]

    h2 = _tc_layer2(h1, s_list, [cnt_nv, cnt_nh, cnt_pp], W2_root, W2_skip,
                    [W2_represents, W2_represented_by,
                     W2_neighboring_vertical, W2_neighboring_horizontal,
                     W2_contains, W2_order, W2_perpendicular])

    return h2[:N]


# speedup vs baseline: 6.3980x; 1.0010x over previous
import functools

import jax
import jax.numpy as jnp
from jax import lax
from jax.experimental import pallas as pl
from jax.experimental.pallas import tpu as pltpu
from jax.experimental.pallas import tpu_sc as plsc

N = 50000
E = 100000

NC = 2
NS = 16
L = 16

TPT = 3136
NPAD = NS * TPT

G = 128
GPT_FULL = 51
GPT_HALF = 27


def _sc_scatter(feat, table, zeros, jobs, table_in_spmem=False):
    njobs = len(jobs)
    rounds = []
    for r in range((njobs + 1) // 2):
        c0 = 2 * r
        c1 = 2 * r + 1 if 2 * r + 1 < njobs else None
        rounds.append((c0, c1))

    idx_inputs = []
    job_meta = []
    any_count = False
    for src, dst, gpt in jobs:
        assert gpt % 3 == 0
        if src is None:
            any_count = True
            job_meta.append((None, len(idx_inputs), gpt))
            idx_inputs.append(dst)
        else:
            job_meta.append((len(idx_inputs), len(idx_inputs) + 1, gpt))
            idx_inputs.append(src)
            idx_inputs.append(dst)

    n_idx = len(idx_inputs)
    max_gpt = max(m[2] for m in job_meta)
    mesh = plsc.VectorSubcoreMesh(core_axis_name="c", subcore_axis_name="s")

    tbl_scratch = ([pltpu.MemorySpace.VMEM_SHARED((NPAD, feat), jnp.float32)]
                   if table_in_spmem else [])
    cnt_scratch = ([pltpu.MemorySpace.VMEM_SHARED((NPAD,), jnp.float32),
                    pltpu.MemorySpace.VMEM((G,), jnp.float32)]
                   if any_count else [])

    @functools.partial(
        pl.kernel,
        out_type=[jax.ShapeDtypeStruct(
            (NPAD, feat) if m[0] is not None else (NPAD,), jnp.float32)
            for m in job_meta],
        mesh=mesh,
        scratch_types=tbl_scratch + cnt_scratch + [
            pltpu.MemorySpace.VMEM_SHARED((NPAD, feat), jnp.float32),
            pltpu.MemorySpace.VMEM((max_gpt, G), jnp.int32),
            pltpu.MemorySpace.VMEM((max_gpt, G), jnp.int32),
            pltpu.MemorySpace.VMEM((3, G, feat), jnp.float32),
            pltpu.SemaphoreType.DMA,
            pltpu.SemaphoreType.DMA,
            pltpu.SemaphoreType.DMA,
            pltpu.SemaphoreType.DMA,
            pltpu.SemaphoreType.DMA,
            pltpu.SemaphoreType.DMA,
        ],
        compiler_params=pltpu.CompilerParams(use_tc_tiling_on_sc=False),
    )
    def run(*refs):
        table_hbm = refs[0]
        zeros_ref = refs[1]
        pos = 2
        if any_count:
            zeros1_ref = refs[pos]
            pos += 1
        idx_refs = refs[pos:pos + n_idx]
        out_refs = refs[pos + n_idx:pos + n_idx + njobs]
        scratch = refs[pos + n_idx + njobs:]
        if table_in_spmem:
            tbl, scratch = scratch[0], scratch[1:]
        if any_count:
            acc1, ones_v = scratch[0], scratch[1]
            scratch = scratch[2:]
        (acc, sidx, didx, rows,
         gs0, gs1, gs2, ss0, ss1, ss2) = scratch
        gsem = (gs0, gs1, gs2)
        ssem = (ss0, ss1, ss2)

        c = lax.axis_index("c")
        s = lax.axis_index("s")
        base = s * TPT

        one16 = jnp.ones((L,), jnp.float32)

        if table_in_spmem:
            pltpu.sync_copy(table_hbm.at[pl.ds(base, TPT)],
                            tbl.at[pl.ds(base, TPT)])
            plsc.subcore_barrier()
            table_ref = tbl
        else:
            table_ref = table_hbm

        if any_count:
            for k in range(G // L):
                ones_v[pl.ds(k * L, L)] = one16

        def emit_job(job_id):
            src_pos, dst_pos, gpt = job_meta[job_id]
            out_ref = out_refs[job_id]
            gbase = s * gpt

            is_count = src_pos is None

            def zero_acc():
                if is_count:
                    pltpu.sync_copy(zeros1_ref, acc1.at[pl.ds(base, TPT)])
                else:
                    pltpu.sync_copy(zeros_ref, acc.at[pl.ds(base, TPT)])

            def issue_gather(g, b):
                pltpu.async_copy(table_ref.at[sidx.at[g]], rows.at[b],
                                 gsem[b])

            def wait_gather(g, b):
                pltpu.make_async_copy(table_ref.at[sidx.at[g]], rows.at[b],
                                      gsem[b]).wait()

            def issue_scatter(g, b):
                pltpu.async_copy(rows.at[b], acc.at[didx.at[g]], ssem[b],
                                 add=True)

            def wait_scatter(g, b):
                pltpu.make_async_copy(rows.at[b], acc.at[didx.at[g]],
                                      ssem[b]).wait()

            def accumulate():
                pltpu.sync_copy(idx_refs[dst_pos].at[pl.ds(gbase, gpt)],
                                didx.at[pl.ds(0, gpt)])
                if src_pos is not None:
                    pltpu.sync_copy(idx_refs[src_pos].at[pl.ds(gbase, gpt)],
                                    sidx.at[pl.ds(0, gpt)])
                    issue_gather(0, 0)
                    issue_gather(1, 1)

                    def pipe(t, carry):
                        for j in range(3):
                            g = 3 * t + j

                            @pl.when(g + 2 < gpt)
                            def _():
                                @pl.when(g >= 1)
                                def _():
                                    wait_scatter(g - 1, (j + 2) % 3)

                                issue_gather(g + 2, (j + 2) % 3)

                            wait_gather(g, j)
                            issue_scatter(g, j)
                        return carry

                    lax.fori_loop(0, gpt // 3, pipe, 0)
                    wait_scatter(gpt - 3, 0)
                    wait_scatter(gpt - 2, 1)
                    wait_scatter(gpt - 1, 2)
                else:
                    def pipe(t, carry):
                        for j in range(3):
                            g = 3 * t + j

                            @pl.when(g >= 3)
                            def _():
                                pltpu.make_async_copy(
                                    ones_v, acc1.at[didx.at[g - 3]],
                                    ssem[j]).wait()

                            pltpu.async_copy(ones_v, acc1.at[didx.at[g]],
                                             ssem[j], add=True)
                        return carry

                    lax.fori_loop(0, gpt // 3, pipe, 0)
                    for j, g in ((0, gpt - 3), (1, gpt - 2), (2, gpt - 1)):
                        pltpu.make_async_copy(ones_v, acc1.at[didx.at[g]],
                                              ssem[j]).wait()

            def flush():
                a = acc1 if is_count else acc
                pltpu.sync_copy(a.at[pl.ds(base, TPT)],
                                out_ref.at[pl.ds(base, TPT)])

            return zero_acc, accumulate, flush

        for c0_job, c1_job in rounds:
            stages0 = emit_job(c0_job)
            stages1 = emit_job(c1_job) if c1_job is not None else None
            pl.when(c == 0)(stages0[0])
            if stages1 is not None:
                pl.when(c == 1)(stages1[0])
            plsc.subcore_barrier()
            pl.when(c == 0)(stages0[1])
            if stages1 is not None:
                pl.when(c == 1)(stages1[1])
            plsc.subcore_barrier()
            pl.when(c == 0)(stages0[2])
            if stages1 is not None:
                pl.when(c == 1)(stages1[2])
            plsc.subcore_barrier()

    if any_count:
        return run(table, zeros, jnp.zeros((TPT,), jnp.float32),
                   *idx_inputs)
    return run(table, zeros, *idx_inputs)


BLK = 1024
GRID = NPAD // BLK


def _row_spec(f):
    return pl.BlockSpec((BLK, f), lambda i: (i, 0))


def _full_spec(fi, fo):
    return pl.BlockSpec((fi, fo), lambda i: (0, 0))


def _tc_layer0(x, s_ct, s_on, w_r, w_ct, w_on):
    def body(x_ref, a_ref, b_ref, wr_ref, wa_ref, wb_ref, o_ref):
        acc = jnp.dot(x_ref[...], wr_ref[...], preferred_element_type=jnp.float32)
        acc += jnp.dot(a_ref[...], wa_ref[...], preferred_element_type=jnp.float32)
        acc += jnp.dot(b_ref[...], wb_ref[...], preferred_element_type=jnp.float32)
        o_ref[...] = acc

    return pl.pallas_call(
        body,
        grid=(GRID,),
        in_specs=[_row_spec(16), _row_spec(16), _row_spec(16),
                  _full_spec(16, 16), _full_spec(16, 16), _full_spec(16, 16)],
        out_specs=_row_spec(16),
        out_shape=jax.ShapeDtypeStruct((NPAD, 16), jnp.float32),
    )(x, s_ct, s_on, w_r, w_ct, w_on)


def _tc_layer1(h, s_ct, s_on, w_r, w_skip, w_ct, w_on):
    def body(h_ref, a_ref, b_ref, wr_ref, ws_ref, wa_ref, wb_ref, o_ref):
        acc = jnp.dot(h_ref[...], wr_ref[...] + ws_ref[...],
                      preferred_element_type=jnp.float32)
        acc += jnp.dot(a_ref[...], wa_ref[...], preferred_element_type=jnp.float32)
        acc += jnp.dot(b_ref[...], wb_ref[...], preferred_element_type=jnp.float32)
        o_ref[...] = acc

    return pl.pallas_call(
        body,
        grid=(GRID,),
        in_specs=[_row_spec(16), _row_spec(16), _row_spec(16),
                  _full_spec(16, 32), _full_spec(16, 32),
                  _full_spec(16, 32), _full_spec(16, 32)],
        out_specs=_row_spec(32),
        out_shape=jax.ShapeDtypeStruct((NPAD, 32), jnp.float32),
    )(h, s_ct, s_on, w_r, w_skip, w_ct, w_on)


MEAN_IDX = (2, 3, 6)


def _tc_layer2(h1, s_list, cnt_list, w_r, w_skip, w_list):
    def body(*refs):
        h_ref = refs[0]
        s_refs = refs[1:9]
        c_refs = refs[9:13]
        wr_ref, ws_ref = refs[13], refs[14]
        w_refs = refs[15:22]
        o_ref = refs[22]
        acc = jnp.dot(h_ref[...], wr_ref[...] + ws_ref[...],
                      preferred_element_type=jnp.float32)
        acc += jnp.dot(s_refs[0][...] + s_refs[1][...], w_refs[0][...],
                       preferred_element_type=jnp.float32)
        cnts = {2: c_refs[0][...], 3: c_refs[1][...],
                6: c_refs[2][...] + c_refs[3][...]}
        for i in range(1, 7):
            m = s_refs[i + 1][...]
            if i in MEAN_IDX:
                m = m / jnp.maximum(cnts[i], 1.0)
            acc += jnp.dot(m, w_refs[i][...],
                           preferred_element_type=jnp.float32)
        o_ref[...] = jnp.maximum(acc, 0.0)

    return pl.pallas_call(
        body,
        grid=(GRID,),
        in_specs=([_row_spec(32)] + [_row_spec(32)] * 8 + [_row_spec(1)] * 4
                  + [_full_spec(32, 64)] * 9),
        out_specs=_row_spec(64),
        out_shape=jax.ShapeDtypeStruct((NPAD, 64), jnp.float32),
    )(h1, *s_list, *cnt_list, w_r, w_skip, *w_list)


def _pad_idx(idx, gpt):
    npad = NS * gpt * G - idx.shape[0]
    fill = N + jnp.arange(npad, dtype=jnp.int32) % (NPAD - N)
    return jnp.concatenate([idx, fill]).reshape(NS * gpt, G)


def _prep_edges(ei, gpt=GPT_FULL):
    return _pad_idx(ei[0], gpt), _pad_idx(ei[1], gpt), gpt


def _pad_w(w):
    return jnp.zeros((16, w.shape[1]), jnp.float32).at[:w.shape[0]].set(w)


def kernel(x, edge_connected_to, edge_ordered_next, edge_represents,
           edge_represented_by, edge_neighboring_vertical,
           edge_neighboring_horizontal, edge_contains, edge_order,
           edge_perpendicular, W0_connected_to, W0_ordered_next, W0_root,
           W1_connected_to, W1_ordered_next, W1_root, W1_skip,
           W2_represents, W2_represented_by, W2_neighboring_vertical,
           W2_neighboring_horizontal, W2_contains, W2_order,
           W2_perpendicular, W2_root, W2_skip):
    x_pad = jnp.zeros((NPAD, 16), jnp.float32).at[:N, :6].set(x)

    ct = _prep_edges(edge_connected_to)
    on = _prep_edges(edge_ordered_next)
    l2 = [edge_represents, edge_represented_by, edge_neighboring_vertical,
          edge_neighboring_horizontal, edge_contains, edge_order,
          edge_perpendicular]
    r0a = (_pad_idx(l2[0][0, :E // 2], GPT_HALF),
           _pad_idx(l2[0][1, :E // 2], GPT_HALF), GPT_HALF)
    r0b = (_pad_idx(l2[0][0, E // 2:], GPT_HALF),
           _pad_idx(l2[0][1, E // 2:], GPT_HALF), GPT_HALF)
    l2_full = [_prep_edges(e) for e in l2[1:]]

    z16 = jnp.zeros((TPT, 16), jnp.float32)
    z32 = jnp.zeros((TPT, 32), jnp.float32)

    cnt_jobs = [(None, _pad_idx(l2[2][1], GPT_FULL), GPT_FULL),
                (None, _pad_idx(l2[3][1], GPT_FULL), GPT_FULL),
                (None, _pad_idx(l2[6][1, :E // 2], GPT_HALF), GPT_HALF),
                (None, _pad_idx(l2[6][1, E // 2:], GPT_HALF), GPT_HALF)]
    jobs1 = [ct, on] + cnt_jobs
    s0_ct, s0_on, cnt_nv, cnt_nh, cnt_ppa, cnt_ppb = _sc_scatter(
        16, x_pad, z16, jobs1, table_in_spmem=True)

    h = _tc_layer0(x_pad, s0_ct, s0_on, _pad_w(W0_root),
                   _pad_w(W0_connected_to), _pad_w(W0_ordered_next))

    s1_ct, s1_on = _sc_scatter(16, h, z16, [ct, on], table_in_spmem=True)

    h1 = _tc_layer1(h, s1_ct, s1_on, W1_root, W1_skip,
                    W1_connected_to, W1_ordered_next)

    jobs3 = [r0a, r0b, l2_full[1], l2_full[0], l2_full[3], l2_full[2],
             l2_full[5], l2_full[4]]
    s2 = _sc_scatter(32, h1, z32, jobs3)
    s_list = [s2[0], s2[1],
              s2[3], s2[2], s2[5], s2[4], s2[7], s2[6]]

    cnt_list = [c.reshape(NPAD, 1)
                for c in (cnt_nv, cnt_nh, cnt_ppa, cnt_ppb)]
    h2 = _tc_layer2(h1, s_list, cnt_list, W2_root, W2_skip,
                    [W2_represents, W2_represented_by,
                     W2_neighboring_vertical, W2_neighboring_horizontal,
                     W2_contains, W2_order, W2_perpendicular])

    return h2[:N]


# speedup vs baseline: 6.4386x; 1.0063x over previous
import functools

import jax
import jax.numpy as jnp
from jax import lax
from jax.experimental import pallas as pl
from jax.experimental.pallas import tpu as pltpu
from jax.experimental.pallas import tpu_sc as plsc

N = 50000
E = 100000

NC = 2
NS = 16
L = 16

TPT = 3136
NPAD = NS * TPT

G = 128
NB = 4
GPT_FULL = 52
GPT_HALF = 28


def _sc_scatter(feat, table, zeros, jobs, table_in_spmem=False):
    njobs = len(jobs)
    rounds = []
    for r in range((njobs + 1) // 2):
        c0 = 2 * r
        c1 = 2 * r + 1 if 2 * r + 1 < njobs else None
        rounds.append((c0, c1))

    idx_inputs = []
    job_meta = []
    any_count = False
    for src, dst, gpt in jobs:
        assert gpt % NB == 0
        if src is None:
            any_count = True
            job_meta.append((None, len(idx_inputs), gpt))
            idx_inputs.append(dst)
        else:
            job_meta.append((len(idx_inputs), len(idx_inputs) + 1, gpt))
            idx_inputs.append(src)
            idx_inputs.append(dst)

    n_idx = len(idx_inputs)
    max_gpt = max(m[2] for m in job_meta)
    mesh = plsc.VectorSubcoreMesh(core_axis_name="c", subcore_axis_name="s")

    tbl_scratch = ([pltpu.MemorySpace.VMEM_SHARED((NPAD, feat), jnp.float32)]
                   if table_in_spmem else [])
    cnt_scratch = ([pltpu.MemorySpace.VMEM_SHARED((NPAD,), jnp.float32),
                    pltpu.MemorySpace.VMEM((G,), jnp.float32)]
                   if any_count else [])

    @functools.partial(
        pl.kernel,
        out_type=[jax.ShapeDtypeStruct(
            (NPAD, feat) if m[0] is not None else (NPAD,), jnp.float32)
            for m in job_meta],
        mesh=mesh,
        scratch_types=tbl_scratch + cnt_scratch + [
            pltpu.MemorySpace.VMEM_SHARED((NPAD, feat), jnp.float32),
            pltpu.MemorySpace.VMEM((max_gpt, G), jnp.int32),
            pltpu.MemorySpace.VMEM((max_gpt, G), jnp.int32),
            pltpu.MemorySpace.VMEM((NB, G, feat), jnp.float32),
        ] + [pltpu.SemaphoreType.DMA] * (2 * NB),

        compiler_params=pltpu.CompilerParams(use_tc_tiling_on_sc=False),
    )
    def run(*refs):
        table_hbm = refs[0]
        zeros_ref = refs[1]
        pos = 2
        if any_count:
            zeros1_ref = refs[pos]
            pos += 1
        idx_refs = refs[pos:pos + n_idx]
        out_refs = refs[pos + n_idx:pos + n_idx + njobs]
        scratch = refs[pos + n_idx + njobs:]
        if table_in_spmem:
            tbl, scratch = scratch[0], scratch[1:]
        if any_count:
            acc1, ones_v = scratch[0], scratch[1]
            scratch = scratch[2:]
        acc, sidx, didx, rows = scratch[:4]
        gsem = scratch[4:4 + NB]
        ssem = scratch[4 + NB:4 + 2 * NB]

        c = lax.axis_index("c")
        s = lax.axis_index("s")
        base = s * TPT

        one16 = jnp.ones((L,), jnp.float32)

        if table_in_spmem:
            pltpu.sync_copy(table_hbm.at[pl.ds(base, TPT)],
                            tbl.at[pl.ds(base, TPT)])
            plsc.subcore_barrier()
            table_ref = tbl
        else:
            table_ref = table_hbm

        if any_count:
            for k in range(G // L):
                ones_v[pl.ds(k * L, L)] = one16

        def emit_job(job_id):
            src_pos, dst_pos, gpt = job_meta[job_id]
            out_ref = out_refs[job_id]
            gbase = s * gpt

            is_count = src_pos is None

            def zero_acc():
                if is_count:
                    pltpu.sync_copy(zeros1_ref, acc1.at[pl.ds(base, TPT)])
                else:
                    pltpu.sync_copy(zeros_ref, acc.at[pl.ds(base, TPT)])

            def issue_gather(g, b):
                pltpu.async_copy(table_ref.at[sidx.at[g]], rows.at[b],
                                 gsem[b])

            def wait_gather(g, b):
                pltpu.make_async_copy(table_ref.at[sidx.at[g]], rows.at[b],
                                      gsem[b]).wait()

            def issue_scatter(g, b):
                pltpu.async_copy(rows.at[b], acc.at[didx.at[g]], ssem[b],
                                 add=True)

            def wait_scatter(g, b):
                pltpu.make_async_copy(rows.at[b], acc.at[didx.at[g]],
                                      ssem[b]).wait()

            def accumulate():
                pltpu.sync_copy(idx_refs[dst_pos].at[pl.ds(gbase, gpt)],
                                didx.at[pl.ds(0, gpt)])
                if src_pos is not None:
                    pltpu.sync_copy(idx_refs[src_pos].at[pl.ds(gbase, gpt)],
                                    sidx.at[pl.ds(0, gpt)])
                    for b in range(NB - 1):
                        issue_gather(b, b)

                    def pipe(t, carry):
                        for j in range(NB):
                            g = NB * t + j

                            @pl.when(g + NB - 1 < gpt)
                            def _():
                                @pl.when(g >= 1)
                                def _():
                                    wait_scatter(g - 1, (j - 1) % NB)

                                issue_gather(g + NB - 1, (j - 1) % NB)

                            wait_gather(g, j)
                            issue_scatter(g, j)
                        return carry

                    lax.fori_loop(0, gpt // NB, pipe, 0)
                    for b in range(NB):
                        wait_scatter(gpt - NB + b, b)
                else:
                    def pipe(t, carry):
                        for j in range(NB):
                            g = NB * t + j

                            @pl.when(g >= NB)
                            def _():
                                pltpu.make_async_copy(
                                    ones_v, acc1.at[didx.at[g - NB]],
                                    ssem[j]).wait()

                            pltpu.async_copy(ones_v, acc1.at[didx.at[g]],
                                             ssem[j], add=True)
                        return carry

                    lax.fori_loop(0, gpt // NB, pipe, 0)
                    for b in range(NB):
                        pltpu.make_async_copy(
                            ones_v, acc1.at[didx.at[gpt - NB + b]],
                            ssem[b]).wait()

            def flush():
                a = acc1 if is_count else acc
                pltpu.sync_copy(a.at[pl.ds(base, TPT)],
                                out_ref.at[pl.ds(base, TPT)])

            return zero_acc, accumulate, flush

        for c0_job, c1_job in rounds:
            stages0 = emit_job(c0_job)
            stages1 = emit_job(c1_job) if c1_job is not None else None
            pl.when(c == 0)(stages0[0])
            if stages1 is not None:
                pl.when(c == 1)(stages1[0])
            plsc.subcore_barrier()
            pl.when(c == 0)(stages0[1])
            if stages1 is not None:
                pl.when(c == 1)(stages1[1])
            plsc.subcore_barrier()
            pl.when(c == 0)(stages0[2])
            if stages1 is not None:
                pl.when(c == 1)(stages1[2])
            plsc.subcore_barrier()

    if any_count:
        return run(table, zeros, jnp.zeros((TPT,), jnp.float32),
                   *idx_inputs)
    return run(table, zeros, *idx_inputs)


BLK = 1024
GRID = NPAD // BLK


def _row_spec(f):
    return pl.BlockSpec((BLK, f), lambda i: (i, 0))


def _full_spec(fi, fo):
    return pl.BlockSpec((fi, fo), lambda i: (0, 0))


def _tc_layer0(x, s_ct, s_on, w_r, w_ct, w_on):
    def body(x_ref, a_ref, b_ref, wr_ref, wa_ref, wb_ref, o_ref):
        acc = jnp.dot(x_ref[...], wr_ref[...], preferred_element_type=jnp.float32)
        acc += jnp.dot(a_ref[...], wa_ref[...], preferred_element_type=jnp.float32)
        acc += jnp.dot(b_ref[...], wb_ref[...], preferred_element_type=jnp.float32)
        o_ref[...] = acc

    return pl.pallas_call(
        body,
        grid=(GRID,),
        in_specs=[_row_spec(16), _row_spec(16), _row_spec(16),
                  _full_spec(16, 16), _full_spec(16, 16), _full_spec(16, 16)],
        out_specs=_row_spec(16),
        out_shape=jax.ShapeDtypeStruct((NPAD, 16), jnp.float32),
    )(x, s_ct, s_on, w_r, w_ct, w_on)


def _tc_layer1(h, s_ct, s_on, w_r, w_skip, w_ct, w_on):
    def body(h_ref, a_ref, b_ref, wr_ref, ws_ref, wa_ref, wb_ref, o_ref):
        acc = jnp.dot(h_ref[...], wr_ref[...] + ws_ref[...],
                      preferred_element_type=jnp.float32)
        acc += jnp.dot(a_ref[...], wa_ref[...], preferred_element_type=jnp.float32)
        acc += jnp.dot(b_ref[...], wb_ref[...], preferred_element_type=jnp.float32)
        o_ref[...] = acc

    return pl.pallas_call(
        body,
        grid=(GRID,),
        in_specs=[_row_spec(16), _row_spec(16), _row_spec(16),
                  _full_spec(16, 32), _full_spec(16, 32),
                  _full_spec(16, 32), _full_spec(16, 32)],
        out_specs=_row_spec(32),
        out_shape=jax.ShapeDtypeStruct((NPAD, 32), jnp.float32),
    )(h, s_ct, s_on, w_r, w_skip, w_ct, w_on)


MEAN_IDX = (2, 3, 6)


def _tc_layer2(h1, s_list, cnt_list, w_r, w_skip, w_list):
    def body(*refs):
        h_ref = refs[0]
        s_refs = refs[1:9]
        c_refs = refs[9:13]
        wr_ref, ws_ref = refs[13], refs[14]
        w_refs = refs[15:22]
        o_ref = refs[22]
        acc = jnp.dot(h_ref[...], wr_ref[...] + ws_ref[...],
                      preferred_element_type=jnp.float32)
        acc += jnp.dot(s_refs[0][...] + s_refs[1][...], w_refs[0][...],
                       preferred_element_type=jnp.float32)
        cnts = {2: c_refs[0][...], 3: c_refs[1][...],
                6: c_refs[2][...] + c_refs[3][...]}
        for i in range(1, 7):
            m = s_refs[i + 1][...]
            if i in MEAN_IDX:
                m = m / jnp.maximum(cnts[i], 1.0)
            acc += jnp.dot(m, w_refs[i][...],
                           preferred_element_type=jnp.float32)
        o_ref[...] = jnp.maximum(acc, 0.0)

    return pl.pallas_call(
        body,
        grid=(GRID,),
        in_specs=([_row_spec(32)] + [_row_spec(32)] * 8 + [_row_spec(1)] * 4
                  + [_full_spec(32, 64)] * 9),
        out_specs=_row_spec(64),
        out_shape=jax.ShapeDtypeStruct((NPAD, 64), jnp.float32),
    )(h1, *s_list, *cnt_list, w_r, w_skip, *w_list)


def _pad_idx(idx, gpt):
    npad = NS * gpt * G - idx.shape[0]
    fill = N + jnp.arange(npad, dtype=jnp.int32) % (NPAD - N)
    return jnp.concatenate([idx, fill]).reshape(NS * gpt, G)


def _prep_edges(ei, gpt=GPT_FULL):
    return _pad_idx(ei[0], gpt), _pad_idx(ei[1], gpt), gpt


def _pad_w(w):
    return jnp.zeros((16, w.shape[1]), jnp.float32).at[:w.shape[0]].set(w)


def kernel(x, edge_connected_to, edge_ordered_next, edge_represents,
           edge_represented_by, edge_neighboring_vertical,
           edge_neighboring_horizontal, edge_contains, edge_order,
           edge_perpendicular, W0_connected_to, W0_ordered_next, W0_root,
           W1_connected_to, W1_ordered_next, W1_root, W1_skip,
           W2_represents, W2_represented_by, W2_neighboring_vertical,
           W2_neighboring_horizontal, W2_contains, W2_order,
           W2_perpendicular, W2_root, W2_skip):
    x_pad = jnp.zeros((NPAD, 16), jnp.float32).at[:N, :6].set(x)

    ct = _prep_edges(edge_connected_to)
    on = _prep_edges(edge_ordered_next)
    l2 = [edge_represents, edge_represented_by, edge_neighboring_vertical,
          edge_neighboring_horizontal, edge_contains, edge_order,
          edge_perpendicular]
    r0a = (_pad_idx(l2[0][0, :E // 2], GPT_HALF),
           _pad_idx(l2[0][1, :E // 2], GPT_HALF), GPT_HALF)
    r0b = (_pad_idx(l2[0][0, E // 2:], GPT_HALF),
           _pad_idx(l2[0][1, E // 2:], GPT_HALF), GPT_HALF)
    l2_full = [_prep_edges(e) for e in l2[1:]]

    z16 = jnp.zeros((TPT, 16), jnp.float32)
    z32 = jnp.zeros((TPT, 32), jnp.float32)

    cnt_jobs = [(None, _pad_idx(l2[2][1], GPT_FULL), GPT_FULL),
                (None, _pad_idx(l2[3][1], GPT_FULL), GPT_FULL),
                (None, _pad_idx(l2[6][1, :E // 2], GPT_HALF), GPT_HALF),
                (None, _pad_idx(l2[6][1, E // 2:], GPT_HALF), GPT_HALF)]
    jobs1 = [ct, on] + cnt_jobs
    s0_ct, s0_on, cnt_nv, cnt_nh, cnt_ppa, cnt_ppb = _sc_scatter(
        16, x_pad, z16, jobs1, table_in_spmem=True)

    h = _tc_layer0(x_pad, s0_ct, s0_on, _pad_w(W0_root),
                   _pad_w(W0_connected_to), _pad_w(W0_ordered_next))

    s1_ct, s1_on = _sc_scatter(16, h, z16, [ct, on], table_in_spmem=True)

    h1 = _tc_layer1(h, s1_ct, s1_on, W1_root, W1_skip,
                    W1_connected_to, W1_ordered_next)

    jobs3 = [r0a, r0b, l2_full[1], l2_full[0], l2_full[3], l2_full[2],
             l2_full[5], l2_full[4]]
    s2 = _sc_scatter(32, h1, z32, jobs3)
    s_list = [s2[0], s2[1],
              s2[3], s2[2], s2[5], s2[4], s2[7], s2[6]]

    cnt_list = [c.reshape(NPAD, 1)
                for c in (cnt_nv, cnt_nh, cnt_ppa, cnt_ppb)]
    h2 = _tc_layer2(h1, s_list, cnt_list, W2_root, W2_skip,
                    [W2_represents, W2_represented_by,
                     W2_neighboring_vertical, W2_neighboring_horizontal,
                     W2_contains, W2_order, W2_perpendicular])

    return h2[:N]
